# Initial kernel scaffold; baseline (speedup 1.0000x reference)
import jax
import jax.numpy as jnp
from jax.experimental import pallas as pl


def kernel(shape_ids, color_ids, edge_index, batch, emb_shape, emb_color, Wl1, bl1, Wr1, Wl2, bl2, Wr2, Wlin, blin):
    raise NotImplementedError("write your pallas kernel here")


# speedup vs baseline: 1.6947x; 1.6947x over previous
import functools

import jax
import jax.numpy as jnp
from jax import lax
from jax.experimental import pallas as pl
from jax.experimental.pallas import tpu as pltpu
from jax.experimental.pallas import tpu_sc as plsc

_N = 100000
_E = 1600000
_EMB = 32
_HID = 64
_G = 1024

_NPAD = 114688
_IDROWS = _NPAD // 128
_EPAD = 1605632
_EROWS = _EPAD // 128

_NREAL = 100352
_DACC = _NREAL + 128
_CH1 = 50176
_CH2 = 25088
_PACC = _G + 128

_MESH = plsc.VectorSubcoreMesh(core_axis_name="c", subcore_axis_name="s")
_f32 = jnp.float32
_i32 = jnp.int32


@functools.partial(
    pl.kernel,
    out_type=jax.ShapeDtypeStruct((4 * _NPAD, 128), _f32),
    mesh=_MESH,
    scratch_types=[
        pltpu.VMEM((128,), _i32),
        pltpu.VMEM((128, 128), _f32),
        pltpu.SemaphoreType.DMA,
    ],
)
def _sc_embed(fused_hbm, fidq_hbm, out_hbm, idx_v, rows_v, sem):
    wid = lax.axis_index("s") * 2 + lax.axis_index("c")

    @pl.loop(0, 112)
    def _chunk(j):
        base = wid * (112 * 128) + j * 128
        pltpu.sync_copy(fidq_hbm.at[pl.ds(base, 128)], idx_v)
        pltpu.async_copy(fused_hbm.at[idx_v], rows_v, sem).wait()
        pltpu.sync_copy(rows_v, out_hbm.at[pl.ds(base, 128), :])


_DROWS = 896


@functools.partial(
    pl.kernel,
    out_type=jax.ShapeDtypeStruct((2, _DROWS, 128), _f32),
    mesh=_MESH,
    scratch_types=[
        pltpu.VMEM((128,), _i32),
        pltpu.VMEM((128,), _i32),
        pltpu.VMEM((128, 128), _f32),
        pltpu.VMEM_SHARED((_DROWS, 128), _f32),
        pltpu.SemaphoreType.DMA,
    ],
)
def _sc_deg(eye_hbm, dlo_hbm, dhi_hbm, z_hbm, out_hbm, lo_v, hi_v, rows_v,
            dacc, sem):
    core = lax.axis_index("c")
    sub = lax.axis_index("s")
    zr = _DROWS // 16
    pltpu.sync_copy(z_hbm.at[pl.ds(sub * zr, zr), :],
                    dacc.at[pl.ds(sub * zr, zr), :])
    plsc.subcore_barrier()

    @pl.loop(0, 392)
    def _blk(j):
        base = (core * 6272 + sub * 392 + j) * 128
        pltpu.sync_copy(dlo_hbm.at[pl.ds(base, 128)], lo_v)
        pltpu.sync_copy(dhi_hbm.at[pl.ds(base, 128)], hi_v)
        pltpu.async_copy(eye_hbm.at[lo_v], rows_v, sem).wait()
        pltpu.sync_copy(rows_v, dacc.at[hi_v], add=True)

    plsc.subcore_barrier()
    pltpu.sync_copy(dacc.at[pl.ds(sub * zr, zr), :],
                    out_hbm.at[core, pl.ds(sub * zr, zr), :])


def _make_sc_agg(logslots, ch, nchunk):
    arows = ch >> logslots
    acc_rows = arows + 128
    zrows = acc_rows // 16
    orows = arows // 16
    out_rows = _NPAD >> logslots

    @functools.partial(
        pl.kernel,
        out_type=jax.ShapeDtypeStruct((out_rows, 128), _f32),
        mesh=_MESH,
        scratch_types=[
            pltpu.VMEM((128,), _i32),
            pltpu.VMEM((128,), _i32),
            pltpu.VMEM((128, 128), _f32),
            pltpu.VMEM_SHARED((acc_rows, 128), _f32),
            pltpu.SemaphoreType.DMA,
        ],
    )
    def _sc_agg(x_hbm, gb_hbm, ib_hbm, z_hbm, out_hbm, gi_v, ii_v,
                rows_v, acc, sem):
        core = lax.axis_index("c")
        sub = lax.axis_index("s")
        for r in range(nchunk):
            chunk = core * nchunk + r
            pltpu.sync_copy(z_hbm.at[pl.ds(sub * zrows, zrows), :],
                            acc.at[pl.ds(sub * zrows, zrows), :])
            plsc.subcore_barrier()

            @pl.loop(0, 784)
            def _blk(j):
                base = (sub * 784 + j) * 128
                pltpu.sync_copy(gb_hbm.at[pl.ds(base, 128)], gi_v)
                pltpu.sync_copy(ib_hbm.at[pl.ds(chunk * _EPAD + base, 128)],
                                ii_v)
                pltpu.async_copy(x_hbm.at[gi_v], rows_v, sem).wait()
                pltpu.sync_copy(rows_v, acc.at[ii_v], add=True)

            plsc.subcore_barrier()
            pltpu.sync_copy(
                acc.at[pl.ds(sub * orows, orows), :],
                out_hbm.at[pl.ds(chunk * arows + sub * orows, orows), :])
            plsc.subcore_barrier()

    return _sc_agg


_sc_agg1 = _make_sc_agg(2, _CH1, 1)
_sc_agg2 = _make_sc_agg(1, _CH2, 2)


@functools.partial(
    pl.kernel,
    out_type=jax.ShapeDtypeStruct((2, _G, 128), _f32),
    mesh=_MESH,
    scratch_types=[
        pltpu.VMEM((128,), _i32),
        pltpu.VMEM((128, 128), _f32),
        pltpu.VMEM_SHARED((_PACC, 128), _f32),
    ],
)
def _sc_pool(x2_hbm, bat_hbm, z_hbm, out_hbm, idx_v, rows_v, pacc):
    core = lax.axis_index("c")
    sub = lax.axis_index("s")
    zr = _PACC // 16
    pltpu.sync_copy(z_hbm.at[pl.ds(sub * zr, zr), :],
                    pacc.at[pl.ds(sub * zr, zr), :])
    plsc.subcore_barrier()

    @pl.loop(0, 28)
    def _blk(j):
        base = (core * 448 + sub * 28 + j) * 128
        pltpu.sync_copy(bat_hbm.at[pl.ds(base, 128)], idx_v)
        pltpu.sync_copy(x2_hbm.at[pl.ds(base, 128), :], rows_v)
        pltpu.sync_copy(rows_v, pacc.at[idx_v], add=True)

    plsc.subcore_barrier()
    pltpu.sync_copy(pacc.at[pl.ds(sub * 64, 64), :],
                    out_hbm.at[core, pl.ds(sub * 64, 64), :])


def _tc_dense(x, agg, degp, WlT, b2d, WrT, col, dup):
    m = x.shape[0]
    d = agg.shape[1]
    bn = 2048
    nout = 2 if dup else 1

    def body(x_ref, a_ref, d_ref, wl_ref, b_ref, wr_ref, c_ref, o_ref):
        d16 = d_ref[0] + d_ref[1]
        deg = jnp.sum(d16, axis=1, keepdims=True) * (1.0 / 16.0)
        r = 1.0 / jnp.maximum(deg, 1.0)
        mean = a_ref[...] * r
        y = (jnp.dot(mean, wl_ref[...], preferred_element_type=_f32)
             + b_ref[...]
             + jnp.dot(x_ref[...], wr_ref[...], preferred_element_type=_f32))
        out = jnp.maximum(y, 0.0) + c_ref[...]
        o_ref[0] = out
        if dup:
            h = out[:, :_HID]
            o_ref[1] = jnp.concatenate([jnp.zeros_like(h), h], axis=1)

    return pl.pallas_call(
        body,
        grid=(m // bn,),
        in_specs=[
            pl.BlockSpec((bn, 128), lambda i: (i, 0)),
            pl.BlockSpec((bn, d), lambda i: (i, 0)),
            pl.BlockSpec((2, bn, 16), lambda i: (0, i, 0)),
            pl.BlockSpec((d, 128), lambda i: (0, 0)),
            pl.BlockSpec((1, 128), lambda i: (0, 0)),
            pl.BlockSpec((128, 128), lambda i: (0, 0)),
            pl.BlockSpec((1, 128), lambda i: (0, 0)),
        ],
        out_specs=pl.BlockSpec((nout, bn, 128), lambda i: (0, i, 0)),
        out_shape=jax.ShapeDtypeStruct((nout, m, 128), _f32),
    )(x, agg, degp, WlT, b2d, WrT, col)


def _tc_head(sp, wT, b2d, fmask, cmask):
    def body(s_ref, w_ref, b_ref, fm_ref, cm_ref, o_ref):
        s = s_ref[0] + s_ref[1]
        cnt = jnp.sum(s * cm_ref[...], axis=1, keepdims=True)
        pooled = s * fm_ref[...] / jnp.maximum(cnt, 1.0)
        o_ref[...] = jnp.dot(pooled, w_ref[...],
                             preferred_element_type=_f32) + b_ref[...]

    return pl.pallas_call(
        body,
        out_shape=jax.ShapeDtypeStruct((_G, 128), _f32),
    )(sp, wT, b2d, fmask, cmask)


def kernel(shape_ids, color_ids, edge_index, batch, emb_shape, emb_color,
           Wl1, bl1, Wr1, Wl2, bl2, Wr2, Wlin, blin):
    pad_n = _NPAD - _N
    pad_e = _EPAD - _E

    fid = jnp.concatenate(
        [shape_ids.astype(_i32) * 16 + color_ids.astype(_i32),
         jnp.zeros((pad_n,), _i32)])
    fidq = (fid[None, :]
            + (1024 * jnp.arange(4, dtype=_i32))[:, None]).reshape(-1)
    bat = jnp.concatenate(
        [batch.astype(_i32),
         _G + (jnp.arange(pad_n, dtype=_i32) & 15)])
    srcp = jnp.concatenate(
        [edge_index[0].astype(_i32), jnp.zeros((pad_e,), _i32)])
    dstp = jnp.concatenate(
        [edge_index[1].astype(_i32),
         _NREAL + (jnp.arange(pad_e, dtype=_i32) & 15)])

    def agg_idx(logslots, ch, nchunks):
        arows = ch >> logslots
        ibs = []
        for c in range(nchunks):
            lo = c * ch
            ok = (dstp >= lo) & (dstp < lo + ch)
            ibs.append(jnp.where(ok, (dstp - lo) >> logslots,
                                 arows + (dstp & 15)))
        return jnp.stack(ibs).reshape(-1)

    gb1 = srcp + (dstp & 3) * _NPAD
    ib1 = agg_idx(2, _CH1, 2)
    gb2 = srcp + (dstp & 1) * _NPAD
    ib2 = agg_idx(1, _CH2, 4)

    fused = (emb_shape[:, None, :] + emb_color[None, :, :]).reshape(-1, _EMB)
    fusedq = jnp.zeros((4, 1024, 128), _f32)
    for k in range(4):
        fusedq = fusedq.at[k, :, 32 * k:32 * k + 32].set(fused)
    fusedq = fusedq.reshape(4096, 128)

    eye = jnp.eye(128, dtype=_f32)
    dlo = dstp & 127
    dhi = dstp >> 7
    zdeg = jnp.zeros((_DROWS, 128), _f32)
    z1 = jnp.zeros(((_CH1 >> 2) + 128, 128), _f32)
    z2 = jnp.zeros(((_CH2 >> 1) + 128, 128), _f32)
    zp = jnp.zeros((_PACC, 128), _f32)

    def padw(WT, h, w):
        return jnp.zeros((h, w), _f32).at[:WT.shape[0], :WT.shape[1]].set(WT)

    col0 = jnp.zeros((1, 128), _f32)
    col64 = jnp.zeros((1, 128), _f32).at[0, _HID].set(1.0)
    fmask = jnp.zeros((1, 128), _f32).at[0, :_HID].set(1.0)
    cmask = jnp.zeros((1, 128), _f32).at[0, _HID].set(1.0)

    xq = _sc_embed(fusedq, fidq)
    x0 = xq[:_NPAD]
    dago = _sc_deg(eye, dlo, dhi, zdeg)
    degv = (dago[0] + dago[1]).reshape(-1)
    degp = jnp.zeros((2, _NPAD, 16), _f32).at[0].set(
        jnp.repeat(degv[:, None], 16, axis=1))

    agg1 = _sc_agg1(xq, gb1, ib1, z1).reshape(_NPAD, _EMB)
    x1d = _tc_dense(x0, agg1, degp, padw(Wl1.T, _EMB, 128),
                    padw(bl1.reshape(1, -1), 1, 128),
                    padw(Wr1.T, 128, 128), col0, True)

    agg2 = _sc_agg2(x1d.reshape(2 * _NPAD, 128), gb2, ib2,
                    z2).reshape(_NPAD, _HID)
    x2 = _tc_dense(x1d[0], agg2, degp, padw(Wl2.T, _HID, 128),
                   padw(bl2.reshape(1, -1), 1, 128),
                   padw(Wr2.T, 128, 128), col64, False)[0]

    sp = _sc_pool(x2, bat, zp)
    out = _tc_head(sp, padw(Wlin.T, 128, 128),
                   padw(blin.reshape(1, -1), 1, 128), fmask, cmask)
    return out[:, :Wlin.shape[0]]
